# baseline (speedup 1.0000x reference)
import functools

import jax
import jax.numpy as jnp
from jax import lax
from jax.experimental import pallas as pl
from jax.experimental.pallas import tpu as pltpu
from jax.experimental.pallas import tpu_sc as plsc

_G = 8
_LIM = 32
_CH = 3
_NW = 32
_PER_W = (_CH * _LIM * _LIM) // _NW
_NPV = (_CH * _G * _G) // 16


def _retina_body(x_hbm, l_hbm, full_hbm, patch_hbm,
                 lv, xv, fv, xch, pv, s_l, s_x, s_ch, s_of):
    w = lax.axis_index("s") * 2 + lax.axis_index("c")
    base = w * _PER_W
    ch = w // (_NPV // _CH)

    c_l = pltpu.async_copy(l_hbm, lv.at[pl.ds(0, 2)], s_l)
    c_x = pltpu.async_copy(x_hbm.at[pl.ds(base, _PER_W)], xv, s_x)

    @pl.when(w < _NPV)
    def _issue_channel():
        pltpu.async_copy(
            x_hbm.at[pl.ds(ch * (_LIM * _LIM), _LIM * _LIM)], xch, s_ch)

    c_l.wait()
    lvec = lv[...]
    vx = 0.5 * ((jnp.clip(lvec[0], -1.0, 1.0) + 1.0) * _LIM)
    vy = 0.5 * ((jnp.clip(lvec[1], -1.0, 1.0) + 1.0) * _LIM)
    ix = vx.astype(jnp.int32)
    iy = vy.astype(jnp.int32)
    fx = ix - (ix.astype(jnp.float32) > vx).astype(jnp.int32) - _G // 2
    fy = iy - (iy.astype(jnp.float32) > vy).astype(jnp.int32) - _G // 2

    lane = lax.iota(jnp.int32, 16)

    c_x.wait()
    for i in range(_PER_W // 16):
        flat = base + i * 16 + lane
        r = (flat % (_LIM * _LIM)) // _LIM
        c = flat % _LIM
        m = (r >= fx) & (r < fx + _G) & (c >= fy) & (c < fy + _G)
        fv[pl.ds(i * 16, 16)] = jnp.where(m, xv[pl.ds(i * 16, 16)], 0.0)
    c_of = pltpu.async_copy(fv, full_hbm.at[pl.ds(base, _PER_W)], s_of)

    @pl.when(w < _NPV)
    def _patch():
        pltpu.make_async_copy(
            x_hbm.at[pl.ds(ch * (_LIM * _LIM), _LIM * _LIM)], xch, s_ch).wait()
        p = (w % (_NPV // _CH)) * 16 + lane
        xi = p // _G
        yi = p % _G
        rows = fx + xi
        cols = fy + yi
        valid = (rows >= 0) & (rows < _LIM) & (cols >= 0) & (cols < _LIM)
        lidx = jnp.clip(rows, 0, _LIM - 1) * _LIM + jnp.clip(cols, 0, _LIM - 1)
        g = plsc.load_gather(xch, [lidx])
        pv[...] = jnp.where(valid, g, 0.0)
        pltpu.sync_copy(pv, patch_hbm.at[pl.ds(w * 16, 16)])

    c_of.wait()


@functools.partial(
    pl.kernel,
    mesh=plsc.VectorSubcoreMesh(core_axis_name="c", subcore_axis_name="s"),
    compiler_params=pltpu.CompilerParams(
        needs_layout_passes=False,
        disable_bounds_checks=True,
        disable_semaphore_checks=True,
        skip_device_barrier=True,
    ),
    out_type=[
        jax.ShapeDtypeStruct((_CH * _LIM * _LIM,), jnp.float32),
        jax.ShapeDtypeStruct((_CH * _G * _G,), jnp.float32),
    ],
    scratch_types=[
        pltpu.VMEM((16,), jnp.float32),
        pltpu.VMEM((_PER_W,), jnp.float32),
        pltpu.VMEM((_PER_W,), jnp.float32),
        pltpu.VMEM((_LIM * _LIM,), jnp.float32),
        pltpu.VMEM((16,), jnp.float32),
        pltpu.SemaphoreType.DMA,
        pltpu.SemaphoreType.DMA,
        pltpu.SemaphoreType.DMA,
        pltpu.SemaphoreType.DMA,
    ],
)
def _retina_sc(*refs):
    _retina_body(*refs)


def kernel(x, l):
    B, C, H, W = x.shape
    full_flat, patch_flat = _retina_sc(x.reshape(-1), l.reshape(-1))
    return (full_flat.reshape(B, C, H, W), patch_flat.reshape(B, C, _G, _G))


# speedup vs baseline: 1.0796x; 1.0796x over previous
import functools

import jax
import jax.numpy as jnp
from jax import lax
from jax.experimental import pallas as pl
from jax.experimental.pallas import tpu as pltpu
from jax.experimental.pallas import tpu_sc as plsc

_G = 8
_LIM = 32
_CH = 3
_NPV = (_CH * _G * _G) // 16


def _origin(lval):
    v = 0.5 * ((jnp.clip(lval, -1.0, 1.0) + 1.0) * _LIM)
    i = v.astype(jnp.int32)
    return i - (i.astype(jnp.float32) > v).astype(jnp.int32) - _G // 2


def _patch_body(x_hbm, l_hbm, patch_hbm, lv, xch, pv, s_l, s_ch):
    w = lax.axis_index("s") * 2 + lax.axis_index("c")

    @pl.when(w < _NPV)
    def _():
        ch = w // (_NPV // _CH)
        c_l = pltpu.async_copy(l_hbm, lv.at[pl.ds(0, 2)], s_l)
        c_ch = pltpu.async_copy(
            x_hbm.at[pl.ds(ch * (_LIM * _LIM), _LIM * _LIM)], xch, s_ch)
        c_l.wait()
        lvec = lv[...]
        fx = _origin(lvec[0])
        fy = _origin(lvec[1])
        lane = lax.iota(jnp.int32, 16)
        p = (w % (_NPV // _CH)) * 16 + lane
        xi = p // _G
        yi = p % _G
        rows = fx + xi
        cols = fy + yi
        valid = (rows >= 0) & (rows < _LIM) & (cols >= 0) & (cols < _LIM)
        lidx = jnp.clip(rows, 0, _LIM - 1) * _LIM + jnp.clip(cols, 0, _LIM - 1)
        c_ch.wait()
        g = plsc.load_gather(xch, [lidx])
        pv[...] = jnp.where(valid, g, 0.0)
        pltpu.sync_copy(pv, patch_hbm.at[pl.ds(w * 16, 16)])


@functools.partial(
    pl.kernel,
    mesh=plsc.VectorSubcoreMesh(core_axis_name="c", subcore_axis_name="s"),
    compiler_params=pltpu.CompilerParams(needs_layout_passes=False),
    out_type=jax.ShapeDtypeStruct((_CH * _G * _G,), jnp.float32),
    scratch_types=[
        pltpu.VMEM((16,), jnp.float32),
        pltpu.VMEM((_LIM * _LIM,), jnp.float32),
        pltpu.VMEM((16,), jnp.float32),
        pltpu.SemaphoreType.DMA,
        pltpu.SemaphoreType.DMA,
    ],
)
def _patch_sc(*refs):
    _patch_body(*refs)


def _full_tc_body(l_ref, x_ref, o_ref):
    fx = _origin(l_ref[0])
    fy = _origin(l_ref[1])
    rr = lax.broadcasted_iota(jnp.int32, (_CH * _LIM, _LIM), 0) % _LIM
    cc = lax.broadcasted_iota(jnp.int32, (_CH * _LIM, _LIM), 1)
    m = (rr >= fx) & (rr < fx + _G) & (cc >= fy) & (cc < fy + _G)
    o_ref[...] = jnp.where(m, x_ref[...], 0.0)


_full_tc = pl.pallas_call(
    _full_tc_body,
    out_shape=jax.ShapeDtypeStruct((_CH * _LIM, _LIM), jnp.float32),
    in_specs=[
        pl.BlockSpec(memory_space=pltpu.SMEM),
        pl.BlockSpec(memory_space=pltpu.VMEM),
    ],
    out_specs=pl.BlockSpec(memory_space=pltpu.VMEM),
)


def kernel(x, l):
    B, C, H, W = x.shape
    lf = l.reshape(-1)
    patch_flat = _patch_sc(x.reshape(-1), lf)
    full2d = _full_tc(lf, x.reshape(C * H, W))
    return (full2d.reshape(B, C, H, W), patch_flat.reshape(B, C, _G, _G))


# speedup vs baseline: 1.1464x; 1.0619x over previous
import functools

import jax
import jax.numpy as jnp
from jax import lax
from jax.experimental import pallas as pl
from jax.experimental.pallas import tpu as pltpu
from jax.experimental.pallas import tpu_sc as plsc

_G = 8
_LIM = 32
_CH = 3
_NPV = (_CH * _G * _G) // 16


def _origin(lval):
    v = 0.5 * ((jnp.clip(lval, -1.0, 1.0) + 1.0) * _LIM)
    i = v.astype(jnp.int32)
    return i - (i.astype(jnp.float32) > v).astype(jnp.int32) - _G // 2


def _patch_body(x_hbm, l_hbm, patch_hbm, lv, xch, pv, s_l, s_ch):
    w = lax.axis_index("s") + lax.axis_index("c") * 0

    @pl.when(w < _NPV)
    def _():
        ch = w // (_NPV // _CH)
        c_l = pltpu.async_copy(l_hbm, lv.at[pl.ds(0, 2)], s_l)
        c_ch = pltpu.async_copy(
            x_hbm.at[pl.ds(ch * (_LIM * _LIM), _LIM * _LIM)], xch, s_ch)
        c_l.wait()
        lvec = lv[...]
        fx = _origin(lvec[0])
        fy = _origin(lvec[1])
        lane = lax.iota(jnp.int32, 16)
        p = (w % (_NPV // _CH)) * 16 + lane
        xi = p // _G
        yi = p % _G
        rows = fx + xi
        cols = fy + yi
        valid = (rows >= 0) & (rows < _LIM) & (cols >= 0) & (cols < _LIM)
        lidx = jnp.clip(rows, 0, _LIM - 1) * _LIM + jnp.clip(cols, 0, _LIM - 1)
        c_ch.wait()
        g = plsc.load_gather(xch, [lidx])
        pv[...] = jnp.where(valid, g, 0.0)
        pltpu.sync_copy(pv, patch_hbm.at[pl.ds(w * 16, 16)])


@functools.partial(
    pl.kernel,
    mesh=plsc.VectorSubcoreMesh(core_axis_name="c", subcore_axis_name="s", num_cores=1),
    compiler_params=pltpu.CompilerParams(needs_layout_passes=False),
    out_type=jax.ShapeDtypeStruct((_CH * _G * _G,), jnp.float32),
    scratch_types=[
        pltpu.VMEM((16,), jnp.float32),
        pltpu.VMEM((_LIM * _LIM,), jnp.float32),
        pltpu.VMEM((16,), jnp.float32),
        pltpu.SemaphoreType.DMA,
        pltpu.SemaphoreType.DMA,
    ],
)
def _patch_sc(*refs):
    _patch_body(*refs)


def _full_tc_body(l_ref, x_ref, o_ref):
    fx = _origin(l_ref[0])
    fy = _origin(l_ref[1])
    rr = lax.broadcasted_iota(jnp.int32, (_CH * _LIM, _LIM), 0) % _LIM
    cc = lax.broadcasted_iota(jnp.int32, (_CH * _LIM, _LIM), 1)
    m = (rr >= fx) & (rr < fx + _G) & (cc >= fy) & (cc < fy + _G)
    o_ref[...] = jnp.where(m, x_ref[...], 0.0)


_full_tc = pl.pallas_call(
    _full_tc_body,
    out_shape=jax.ShapeDtypeStruct((_CH * _LIM, _LIM), jnp.float32),
    in_specs=[
        pl.BlockSpec(memory_space=pltpu.SMEM),
        pl.BlockSpec(memory_space=pltpu.VMEM),
    ],
    out_specs=pl.BlockSpec(memory_space=pltpu.VMEM),
)


def kernel(x, l):
    B, C, H, W = x.shape
    lf = l.reshape(-1)
    patch_flat = _patch_sc(x.reshape(-1), lf)
    full2d = _full_tc(lf, x.reshape(C * H, W))
    return (full2d.reshape(B, C, H, W), patch_flat.reshape(B, C, _G, _G))


# speedup vs baseline: 1.1525x; 1.0053x over previous
import functools

import jax
import jax.numpy as jnp
from jax import lax
from jax.experimental import pallas as pl
from jax.experimental.pallas import tpu as pltpu
from jax.experimental.pallas import tpu_sc as plsc

_G = 8
_LIM = 32
_CH = 3
_N = _CH * _LIM * _LIM
_NPV = (_CH * _G * _G) // 16


def _origin(lval):
    v = 0.5 * ((jnp.clip(lval, -1.0, 1.0) + 1.0) * _LIM)
    i = v.astype(jnp.int32)
    return i - (i.astype(jnp.float32) > v).astype(jnp.int32) - _G // 2


def _patch_body(x_hbm, l_hbm, patch_hbm, lv, xch, pv, s_l, s_ch):
    c_l = pltpu.async_copy(l_hbm, lv.at[pl.ds(0, 2)], s_l)
    c_x = pltpu.async_copy(x_hbm, xch, s_ch)
    c_l.wait()
    lvec = lv[...]
    fx = _origin(lvec[0])
    fy = _origin(lvec[1])
    lane = lax.iota(jnp.int32, 16)
    c_x.wait()
    for j in range(_NPV):
        p = (j % (_NPV // _CH)) * 16 + lane
        ch = j // (_NPV // _CH)
        xi = p // _G
        yi = p % _G
        rows = fx + xi
        cols = fy + yi
        valid = (rows >= 0) & (rows < _LIM) & (cols >= 0) & (cols < _LIM)
        lidx = (ch * _LIM * _LIM
                + jnp.clip(rows, 0, _LIM - 1) * _LIM
                + jnp.clip(cols, 0, _LIM - 1))
        g = plsc.load_gather(xch, [lidx])
        pv[pl.ds(j * 16, 16)] = jnp.where(valid, g, 0.0)
    pltpu.sync_copy(pv, patch_hbm)


@functools.partial(
    pl.kernel,
    mesh=plsc.VectorSubcoreMesh(
        core_axis_name="c", subcore_axis_name="s",
        num_cores=1, num_subcores=1),
    compiler_params=pltpu.CompilerParams(needs_layout_passes=False),
    out_type=jax.ShapeDtypeStruct((_CH * _G * _G,), jnp.float32),
    scratch_types=[
        pltpu.VMEM((16,), jnp.float32),
        pltpu.VMEM((_N,), jnp.float32),
        pltpu.VMEM((_CH * _G * _G,), jnp.float32),
        pltpu.SemaphoreType.DMA,
        pltpu.SemaphoreType.DMA,
    ],
)
def _patch_sc(*refs):
    _patch_body(*refs)


def _full_tc_body(l_ref, x_ref, o_ref):
    fx = _origin(l_ref[0])
    fy = _origin(l_ref[1])
    rr = lax.broadcasted_iota(jnp.int32, (_CH * _LIM, _LIM), 0) % _LIM
    cc = lax.broadcasted_iota(jnp.int32, (_CH * _LIM, _LIM), 1)
    m = (rr >= fx) & (rr < fx + _G) & (cc >= fy) & (cc < fy + _G)
    o_ref[...] = jnp.where(m, x_ref[...], 0.0)


_full_tc = pl.pallas_call(
    _full_tc_body,
    out_shape=jax.ShapeDtypeStruct((_CH * _LIM, _LIM), jnp.float32),
    in_specs=[
        pl.BlockSpec(memory_space=pltpu.SMEM),
        pl.BlockSpec(memory_space=pltpu.VMEM),
    ],
    out_specs=pl.BlockSpec(memory_space=pltpu.VMEM),
)


def kernel(x, l):
    B, C, H, W = x.shape
    lf = l.reshape(-1)
    patch_flat = _patch_sc(x.reshape(-1), lf)
    full2d = _full_tc(lf, x.reshape(C * H, W))
    return (full2d.reshape(B, C, H, W), patch_flat.reshape(B, C, _G, _G))
